# baseline (speedup 1.0000x reference)
import jax
import jax.numpy as jnp
from jax import lax
from jax.experimental import pallas as pl
from jax.experimental.pallas import tpu as pltpu
from jax.experimental.pallas import tpu_sc as plsc

VALUES_PER_CHANNEL = 256
C = 3
D = 256
L = 16
K = C * L
NB = 4
NW = 32
R = 1792
W = 224
KB = W // L
NS = 14
SC_ = 56


def _build():
    rows_per_w = R // NW
    G = rows_per_w * KB
    assert G == NS * SC_ and SC_ % NB == 0

    mesh = plsc.VectorSubcoreMesh(core_axis_name="c", subcore_axis_name="s")

    @pl.kernel(
        out_type=jax.ShapeDtypeStruct((R, W, C * D), jnp.float32),
        mesh=mesh,
        scratch_types=[
            [pltpu.VMEM((SC_, K), jnp.int32) for _ in range(2)],
            [pltpu.VMEM((L, C * D), jnp.float32) for _ in range(NB)],
            [pltpu.SemaphoreType.DMA for _ in range(2)],
            [pltpu.SemaphoreType.DMA for _ in range(NB)],
            [pltpu.SemaphoreType.DMA for _ in range(NB)],
        ],
    )
    def k(idx_hbm, table_hbm, out_hbm, ibufs, bufs, isems, gsems, wsems):
        cid = lax.axis_index("c")
        sid = lax.axis_index("s")
        wid = sid * 2 + cid
        row0 = wid * rows_per_w

        def idx_load(s, sp):
            return pltpu.make_async_copy(idx_hbm.at[wid, s], ibufs[sp],
                                         isems[sp])

        def gathers(h, t, b, ib):
            return [
                pltpu.make_async_copy(
                    table_hbm.at[ib.at[t, pl.ds(c * L, L)]],
                    bufs[b].at[:, pl.ds(c * D, D)],
                    gsems[b])
                for c in range(C)
            ]

        def writes(h, b):
            r = row0 + h // KB
            w0 = (h % KB) * L
            return [
                pltpu.make_async_copy(
                    bufs[b],
                    out_hbm.at[r, pl.ds(w0, L), :],
                    wsems[b])
            ]

        idx_load(0, 0).start()
        idx_load(1, 1).start()

        @pl.loop(0, NS, step=2)
        def _(so):
            for sp in range(2):
                s = so + sp
                ib = ibufs[sp]
                h0 = s * SC_
                idx_load(s, sp).wait()

                @pl.loop(0, SC_)
                def _(t):
                    ib[t, pl.ds(L, L)] = ib[t, pl.ds(L, L)] + VALUES_PER_CHANNEL
                    ib[t, pl.ds(2 * L, L)] = (ib[t, pl.ds(2 * L, L)]
                                              + 2 * VALUES_PER_CHANNEL)

                for t in range(2):
                    h = h0 + t

                    @pl.when(h >= NB)
                    def _():
                        for d_ in writes(h - NB, t):
                            d_.wait()
                    for d_ in gathers(h, t, t, ib):
                        d_.start()

                @pl.loop(0, SC_, step=NB)
                def _(to):
                    for b in range(NB):
                        t = to + b
                        h = h0 + t

                        @pl.when(t + 2 < SC_)
                        def _():
                            b2 = (b + 2) % NB

                            @pl.when(h >= 2)
                            def _():
                                for d_ in writes(h - 2, b2):
                                    d_.wait()
                            for d_ in gathers(h + 2, t + 2, b2, ib):
                                d_.start()

                        for d_ in gathers(h, t, b, ib):
                            d_.wait()
                        for d_ in writes(h, b):
                            d_.start()

                @pl.when(s + 2 < NS)
                def _():
                    idx_load(s + 2, sp).start()

        for t in range(NB):
            h = G - NB + t
            for d_ in writes(h, h % NB):
                d_.wait()

    return k


def kernel(inputs, table):
    idx = inputs.reshape(R, KB, L, C).transpose(0, 1, 3, 2)
    idx = idx.reshape(NW, NS, SC_, K)
    out = _build()(idx, table)
    return out.reshape(inputs.shape[:1] + (224, 224, 768))
